# baseline (speedup 1.0000x reference)
import functools

import jax
import jax.numpy as jnp
from jax import lax
from jax.experimental import pallas as pl
from jax.experimental.pallas import tpu as pltpu
from jax.experimental.pallas import tpu_sc as plsc

N = 10000
E = 160000
MAXN = 9993

NC = 2
NS = 16
NW = NC * NS
CHUNK = 128
NCHUNK = E // CHUNK
NPAD = 10240
ROWS_PER_TILE = NPAD // NS
GK = 8
CHUNKS_PER_TILE = 40
EPAD = NW * CHUNKS_PER_TILE * CHUNK
CNT0, CNT1 = 24, 56
CNTMAX = max(CNT0, CNT1)


def _make_edge_pass(F):
  mesh = plsc.VectorSubcoreMesh(core_axis_name="c", subcore_axis_name="s")

  @functools.partial(
      pl.kernel,
      mesh=mesh,
      out_type=jax.ShapeDtypeStruct((NC, NPAD, F), jnp.float32),
      scratch_types=[
          pltpu.VMEM((CNTMAX, CHUNK), jnp.int32),
          pltpu.VMEM((CNTMAX, CHUNK), jnp.int32),
          pltpu.VMEM((2, GK, CHUNK, F), jnp.float32),
          pltpu.VMEM((ROWS_PER_TILE, F), jnp.float32),
          pltpu.VMEM_SHARED((NPAD, F), jnp.float32),
          pltpu.SemaphoreType.DMA,
          pltpu.SemaphoreType.DMA,
      ],
      compiler_params=pltpu.CompilerParams(use_tc_tiling_on_sc=False),
  )
  def edge_pass(y_hbm, src_hbm, dst_hbm, zeros_hbm, out_hbm,
                sidx, didx, rows, stage, acc, semg, sems):
    c = lax.axis_index("c")
    s = lax.axis_index("s")
    roff = s * ROWS_PER_TILE
    start = lax.select(c == 0, s * CNT0, NS * CNT0 + s * CNT1)
    ngroup = lax.select(c == 0, CNT0 // GK, CNT1 // GK)

    pltpu.async_copy(src_hbm.at[pl.ds(start, CNTMAX)], sidx, sems)
    pltpu.async_copy(dst_hbm.at[pl.ds(start, CNTMAX)], didx, sems)
    pltpu.sync_copy(zeros_hbm, stage)
    pltpu.sync_copy(stage, acc.at[pl.ds(roff, ROWS_PER_TILE)])
    pltpu.make_async_copy(src_hbm.at[pl.ds(0, CNTMAX)], sidx, sems).wait()
    pltpu.make_async_copy(src_hbm.at[pl.ds(0, CNTMAX)], didx, sems).wait()
    plsc.subcore_barrier()

    def fire_gathers(g, bank):
      for j in range(GK):
        pltpu.async_copy(y_hbm.at[sidx.at[g * GK + j]], rows.at[bank, j],
                         semg)

    def drain_gathers(bank):
      for j in range(GK):
        pltpu.make_async_copy(y_hbm.at[sidx.at[0]], rows.at[bank, j],
                              semg).wait()

    def drain_scatters(bank):
      for j in range(GK):
        pltpu.make_async_copy(rows.at[bank, j], acc.at[didx.at[0]],
                              sems).wait()

    fire_gathers(0, 0)

    def group(g, carry):
      bank = lax.rem(g, 2)
      drain_gathers(bank)

      @pl.when(g + 1 < ngroup)
      def _():
        fire_gathers(g + 1, 1 - bank)

      for j in range(GK):
        pltpu.async_copy(rows.at[bank, j], acc.at[didx.at[g * GK + j]], sems,
                         add=True)
      drain_scatters(bank)
      return carry

    lax.fori_loop(0, ngroup, group, 0)
    plsc.subcore_barrier()

    pltpu.sync_copy(acc.at[pl.ds(roff, ROWS_PER_TILE)], stage)
    pltpu.sync_copy(stage, out_hbm.at[c, pl.ds(roff, ROWS_PER_TILE)])

  return edge_pass


def _relu6(x):
  return jnp.clip(x, 0.0, 6.0)


def _entry_body(feat_ref, w_ref, b_ref, out_ref):
  out_ref[...] = _relu6(
      jnp.dot(feat_ref[...], w_ref[...], preferred_element_type=jnp.float32)
      + b_ref[...])


def _deg_body(degp_ref, x0_ref, w_ref, dinv_ref, y_ref):
  deg = degp_ref[0, :N, 0:1] + degp_ref[1, :N, 0:1] + 1.0
  dinv = lax.rsqrt(jnp.maximum(deg, 1e-12))
  dinv_ref[...] = dinv
  y_ref[...] = jnp.dot(x0_ref[...], w_ref[...],
                       preferred_element_type=jnp.float32) * dinv


def _layer_body(zp_ref, y_ref, dinv_ref, b_ref, wn_ref, yn_ref):
  h = _relu6((zp_ref[0, :N] + zp_ref[1, :N] + y_ref[...]) * dinv_ref[...]
             + b_ref[...])
  yn_ref[...] = jnp.dot(h, wn_ref[...],
                        preferred_element_type=jnp.float32) * dinv_ref[...]


def _head_body(zp_ref, y_ref, dinv_ref, bg3_ref, ws1_ref, bs1_ref, ws2_ref,
               bs2_ref, wt1a_ref, wt1b_ref, bt1_ref, wt2_ref, bt2_ref,
               mask_ref, sprob_ref, sidx_ref, tprob_ref, tidx_ref):
  x = _relu6((zp_ref[0, :N] + zp_ref[1, :N] + y_ref[...]) * dinv_ref[...]
             + bg3_ref[...])
  sh = _relu6(jnp.dot(x, ws1_ref[...], preferred_element_type=jnp.float32)
              + bs1_ref[...])
  sl = jnp.dot(sh, ws2_ref[...], preferred_element_type=jnp.float32) \
      + bs2_ref[...]
  sp = jnp.exp(sl - jnp.max(sl))
  sp = sp / jnp.sum(sp)
  m = mask_ref[...] > 0.0
  sprob_ref[...] = jnp.where(m, 0.0, sp)
  rows = lax.broadcasted_iota(jnp.int32, (N, 1), 0)
  sm = jnp.where(m, -1.0, sp)
  smx = jnp.max(sm)
  sidx = jnp.min(jnp.where(sm == smx, rows, N))
  sidx_ref[...] = jnp.reshape(sidx, (1, 1))
  xs = jnp.sum(jnp.where(rows == sidx, x, 0.0), axis=0, keepdims=True)
  th = _relu6(jnp.dot(x, wt1a_ref[...], preferred_element_type=jnp.float32)
              + jnp.dot(xs, wt1b_ref[...], preferred_element_type=jnp.float32)
              + bt1_ref[...])
  tl = jnp.dot(th, wt2_ref[...], preferred_element_type=jnp.float32) \
      + bt2_ref[...]
  tp = jnp.exp(tl - jnp.max(tl))
  tp = tp / jnp.sum(tp)
  tmask = rows < MAXN
  tprob_ref[...] = jnp.where(tmask, tp, 0.0)
  tmx = jnp.max(jnp.where(tmask, tp, -1.0))
  tidx = jnp.min(jnp.where((tp == tmx) & tmask, rows, N))
  tidx_ref[...] = jnp.reshape(tidx, (1, 1))


def _tc_call(body, out_shapes):
  return pl.pallas_call(
      body,
      out_shape=out_shapes,
  )


def kernel(feat, edge_index, mask_candidate_set, W0, b0, Wg1, bg1, Wg2, bg2,
           Wg3, bg3, Ws1, bs1, Ws2, bs2, Wt1, bt1, Wt2, bt2):
  f32 = jnp.float32
  src = jnp.concatenate(
      [edge_index[0].astype(jnp.int32),
       jnp.zeros((EPAD - E,), jnp.int32)]).reshape(EPAD // CHUNK, CHUNK)
  dst = jnp.concatenate(
      [edge_index[1].astype(jnp.int32),
       jnp.full((EPAD - E,), N, jnp.int32)]).reshape(EPAD // CHUNK, CHUNK)

  x0 = _tc_call(_entry_body, jax.ShapeDtypeStruct((N, 8), f32))(
      feat, W0, b0.reshape(1, 8))

  ones8 = jnp.ones((N, 8), f32)
  degp = _make_edge_pass(8)(ones8, src, dst, jnp.zeros((ROWS_PER_TILE, 8), f32))

  dinv, y1 = _tc_call(
      _deg_body,
      (jax.ShapeDtypeStruct((N, 1), f32), jax.ShapeDtypeStruct((N, 16), f32)),
  )(degp, x0, Wg1)

  z1 = _make_edge_pass(16)(y1, src, dst, jnp.zeros((ROWS_PER_TILE, 16), f32))
  y2 = _tc_call(_layer_body, jax.ShapeDtypeStruct((N, 24), f32))(
      z1, y1, dinv, bg1.reshape(1, 16), Wg2)

  z2 = _make_edge_pass(24)(y2, src, dst, jnp.zeros((ROWS_PER_TILE, 24), f32))
  y3 = _tc_call(_layer_body, jax.ShapeDtypeStruct((N, 32), f32))(
      z2, y2, dinv, bg2.reshape(1, 24), Wg3)

  z3 = _make_edge_pass(32)(y3, src, dst, jnp.zeros((ROWS_PER_TILE, 32), f32))

  sprob, sidx, tprob, tidx = _tc_call(
      _head_body,
      (jax.ShapeDtypeStruct((N, 1), f32),
       jax.ShapeDtypeStruct((1, 1), jnp.int32),
       jax.ShapeDtypeStruct((N, 1), f32),
       jax.ShapeDtypeStruct((1, 1), jnp.int32)),
  )(z3, y3, dinv, bg3.reshape(1, 32), Ws1, bs1.reshape(1, 16), Ws2,
    bs2.reshape(1, 1), Wt1[:32], Wt1[32:], bt1.reshape(1, 24), Wt2,
    bt2.reshape(1, 1), mask_candidate_set.astype(f32).reshape(N, 1))

  return sprob, sidx[0, 0], tprob, tidx[0, 0]


# speedup vs baseline: 1.0602x; 1.0602x over previous
import functools

import jax
import jax.numpy as jnp
from jax import lax
from jax.experimental import pallas as pl
from jax.experimental.pallas import tpu as pltpu
from jax.experimental.pallas import tpu_sc as plsc

N = 10000
E = 160000
MAXN = 9993

NC = 2
NS = 16
NW = NC * NS
CHUNK = 128
NCHUNK = E // CHUNK
NPAD = 10240
ROWS_PER_TILE = NPAD // NS
GK = 8
CHUNKS_PER_TILE = 40
EPAD = NW * CHUNKS_PER_TILE * CHUNK
CNT0, CNT1 = 56, 24
CNTMAX = max(CNT0, CNT1)


def _make_edge_pass(F):
  mesh = plsc.VectorSubcoreMesh(core_axis_name="c", subcore_axis_name="s")

  @functools.partial(
      pl.kernel,
      mesh=mesh,
      out_type=jax.ShapeDtypeStruct((NC, NPAD, F), jnp.float32),
      scratch_types=[
          pltpu.VMEM((CNTMAX, CHUNK), jnp.int32),
          pltpu.VMEM((CNTMAX, CHUNK), jnp.int32),
          pltpu.VMEM((2, GK, CHUNK, F), jnp.float32),
          pltpu.VMEM((ROWS_PER_TILE, F), jnp.float32),
          pltpu.VMEM_SHARED((NPAD, F), jnp.float32),
          pltpu.SemaphoreType.DMA,
          pltpu.SemaphoreType.DMA,
      ],
      compiler_params=pltpu.CompilerParams(use_tc_tiling_on_sc=False),
  )
  def edge_pass(y_hbm, src_hbm, dst_hbm, zeros_hbm, out_hbm,
                sidx, didx, rows, stage, acc, semg, sems):
    c = lax.axis_index("c")
    s = lax.axis_index("s")
    roff = s * ROWS_PER_TILE
    start = lax.select(c == 0, s * CNT0, NS * CNT0 + s * CNT1)
    ngroup = lax.select(c == 0, CNT0 // GK, CNT1 // GK)

    pltpu.async_copy(src_hbm.at[pl.ds(start, CNTMAX)], sidx, sems)
    pltpu.async_copy(dst_hbm.at[pl.ds(start, CNTMAX)], didx, sems)
    pltpu.sync_copy(zeros_hbm, stage)
    pltpu.sync_copy(stage, acc.at[pl.ds(roff, ROWS_PER_TILE)])
    pltpu.make_async_copy(src_hbm.at[pl.ds(0, CNTMAX)], sidx, sems).wait()
    pltpu.make_async_copy(src_hbm.at[pl.ds(0, CNTMAX)], didx, sems).wait()
    plsc.subcore_barrier()

    def fire_gathers(g, bank):
      for j in range(GK):
        pltpu.async_copy(y_hbm.at[sidx.at[g * GK + j]], rows.at[bank, j],
                         semg)

    def drain_gathers(bank):
      for j in range(GK):
        pltpu.make_async_copy(y_hbm.at[sidx.at[0]], rows.at[bank, j],
                              semg).wait()

    def drain_scatters(bank):
      for j in range(GK):
        pltpu.make_async_copy(rows.at[bank, j], acc.at[didx.at[0]],
                              sems).wait()

    fire_gathers(0, 0)

    def group(g, carry):
      bank = lax.rem(g, 2)
      drain_gathers(bank)

      @pl.when(g + 1 < ngroup)
      def _():
        fire_gathers(g + 1, 1 - bank)

      for j in range(GK):
        pltpu.async_copy(rows.at[bank, j], acc.at[didx.at[g * GK + j]], sems,
                         add=True)
      drain_scatters(bank)
      return carry

    lax.fori_loop(0, ngroup, group, 0)
    plsc.subcore_barrier()

    pltpu.sync_copy(acc.at[pl.ds(roff, ROWS_PER_TILE)], stage)
    pltpu.sync_copy(stage, out_hbm.at[c, pl.ds(roff, ROWS_PER_TILE)])

  return edge_pass


def _relu6(x):
  return jnp.clip(x, 0.0, 6.0)


def _entry_body(feat_ref, w_ref, b_ref, out_ref):
  out_ref[...] = _relu6(
      jnp.dot(feat_ref[...], w_ref[...], preferred_element_type=jnp.float32)
      + b_ref[...])


def _deg_body(degp_ref, x0_ref, w_ref, dinv_ref, y_ref):
  deg = degp_ref[0, :N, 0:1] + degp_ref[1, :N, 0:1] + 1.0
  dinv = lax.rsqrt(jnp.maximum(deg, 1e-12))
  dinv_ref[...] = dinv
  y_ref[...] = jnp.dot(x0_ref[...], w_ref[...],
                       preferred_element_type=jnp.float32) * dinv


def _layer_body(zp_ref, y_ref, dinv_ref, b_ref, wn_ref, yn_ref):
  h = _relu6((zp_ref[0, :N] + zp_ref[1, :N] + y_ref[...]) * dinv_ref[...]
             + b_ref[...])
  yn_ref[...] = jnp.dot(h, wn_ref[...],
                        preferred_element_type=jnp.float32) * dinv_ref[...]


def _head_body(zp_ref, y_ref, dinv_ref, bg3_ref, ws1_ref, bs1_ref, ws2_ref,
               bs2_ref, wt1a_ref, wt1b_ref, bt1_ref, wt2_ref, bt2_ref,
               mask_ref, sprob_ref, sidx_ref, tprob_ref, tidx_ref):
  x = _relu6((zp_ref[0, :N] + zp_ref[1, :N] + y_ref[...]) * dinv_ref[...]
             + bg3_ref[...])
  sh = _relu6(jnp.dot(x, ws1_ref[...], preferred_element_type=jnp.float32)
              + bs1_ref[...])
  sl = jnp.dot(sh, ws2_ref[...], preferred_element_type=jnp.float32) \
      + bs2_ref[...]
  sp = jnp.exp(sl - jnp.max(sl))
  sp = sp / jnp.sum(sp)
  m = mask_ref[...] > 0.0
  sprob_ref[...] = jnp.where(m, 0.0, sp)
  rows = lax.broadcasted_iota(jnp.int32, (N, 1), 0)
  sm = jnp.where(m, -1.0, sp)
  smx = jnp.max(sm)
  sidx = jnp.min(jnp.where(sm == smx, rows, N))
  sidx_ref[...] = jnp.reshape(sidx, (1, 1))
  xs = jnp.sum(jnp.where(rows == sidx, x, 0.0), axis=0, keepdims=True)
  th = _relu6(jnp.dot(x, wt1a_ref[...], preferred_element_type=jnp.float32)
              + jnp.dot(xs, wt1b_ref[...], preferred_element_type=jnp.float32)
              + bt1_ref[...])
  tl = jnp.dot(th, wt2_ref[...], preferred_element_type=jnp.float32) \
      + bt2_ref[...]
  tp = jnp.exp(tl - jnp.max(tl))
  tp = tp / jnp.sum(tp)
  tmask = rows < MAXN
  tprob_ref[...] = jnp.where(tmask, tp, 0.0)
  tmx = jnp.max(jnp.where(tmask, tp, -1.0))
  tidx = jnp.min(jnp.where((tp == tmx) & tmask, rows, N))
  tidx_ref[...] = jnp.reshape(tidx, (1, 1))


def _tc_call(body, out_shapes):
  return pl.pallas_call(
      body,
      out_shape=out_shapes,
  )


def kernel(feat, edge_index, mask_candidate_set, W0, b0, Wg1, bg1, Wg2, bg2,
           Wg3, bg3, Ws1, bs1, Ws2, bs2, Wt1, bt1, Wt2, bt2):
  f32 = jnp.float32
  src = jnp.concatenate(
      [edge_index[0].astype(jnp.int32),
       jnp.zeros((EPAD - E,), jnp.int32)]).reshape(EPAD // CHUNK, CHUNK)
  dst = jnp.concatenate(
      [edge_index[1].astype(jnp.int32),
       jnp.full((EPAD - E,), N, jnp.int32)]).reshape(EPAD // CHUNK, CHUNK)

  x0 = _tc_call(_entry_body, jax.ShapeDtypeStruct((N, 8), f32))(
      feat, W0, b0.reshape(1, 8))

  ones8 = jnp.ones((N, 8), f32)
  degp = _make_edge_pass(8)(ones8, src, dst, jnp.zeros((ROWS_PER_TILE, 8), f32))

  dinv, y1 = _tc_call(
      _deg_body,
      (jax.ShapeDtypeStruct((N, 1), f32), jax.ShapeDtypeStruct((N, 16), f32)),
  )(degp, x0, Wg1)

  z1 = _make_edge_pass(16)(y1, src, dst, jnp.zeros((ROWS_PER_TILE, 16), f32))
  y2 = _tc_call(_layer_body, jax.ShapeDtypeStruct((N, 24), f32))(
      z1, y1, dinv, bg1.reshape(1, 16), Wg2)

  z2 = _make_edge_pass(24)(y2, src, dst, jnp.zeros((ROWS_PER_TILE, 24), f32))
  y3 = _tc_call(_layer_body, jax.ShapeDtypeStruct((N, 32), f32))(
      z2, y2, dinv, bg2.reshape(1, 24), Wg3)

  z3 = _make_edge_pass(32)(y3, src, dst, jnp.zeros((ROWS_PER_TILE, 32), f32))

  sprob, sidx, tprob, tidx = _tc_call(
      _head_body,
      (jax.ShapeDtypeStruct((N, 1), f32),
       jax.ShapeDtypeStruct((1, 1), jnp.int32),
       jax.ShapeDtypeStruct((N, 1), f32),
       jax.ShapeDtypeStruct((1, 1), jnp.int32)),
  )(z3, y3, dinv, bg3.reshape(1, 32), Ws1, bs1.reshape(1, 16), Ws2,
    bs2.reshape(1, 1), Wt1[:32], Wt1[32:], bt1.reshape(1, 24), Wt2,
    bt2.reshape(1, 1), mask_candidate_set.astype(f32).reshape(N, 1))

  return sprob, sidx[0, 0], tprob, tidx[0, 0]


# speedup vs baseline: 1.2964x; 1.2227x over previous
import functools

import jax
import jax.numpy as jnp
from jax import lax
from jax.experimental import pallas as pl
from jax.experimental.pallas import tpu as pltpu
from jax.experimental.pallas import tpu_sc as plsc

N = 10000
E = 160000
MAXN = 9993

NC = 2
NS = 16
NW = NC * NS
CHUNK = 128
NCHUNK = E // CHUNK
NPAD = 10240
ROWS_PER_TILE = NPAD // NS
GK = 8
CHUNKS_PER_TILE = 40
EPAD = NW * CHUNKS_PER_TILE * CHUNK
CNT0, CNT1 = 40, 40
CNTMAX = max(CNT0, CNT1)
YCOPY = 640


def _make_edge_pass(F):
  mesh = plsc.VectorSubcoreMesh(core_axis_name="c", subcore_axis_name="s")

  @functools.partial(
      pl.kernel,
      mesh=mesh,
      out_type=jax.ShapeDtypeStruct((NC, NPAD, F), jnp.float32),
      scratch_types=[
          pltpu.VMEM((CNTMAX, CHUNK), jnp.int32),
          pltpu.VMEM((CNTMAX, CHUNK), jnp.int32),
          pltpu.VMEM((2, GK, CHUNK, F), jnp.float32),
          pltpu.VMEM((ROWS_PER_TILE, F), jnp.float32),
          pltpu.VMEM_SHARED((NPAD, F), jnp.float32),
          pltpu.VMEM_SHARED((N, F), jnp.float32) if F <= 24 else None,
          pltpu.SemaphoreType.DMA,
          pltpu.SemaphoreType.DMA,
      ],
      compiler_params=pltpu.CompilerParams(use_tc_tiling_on_sc=False),
  )
  def edge_pass(y_hbm, src_hbm, dst_hbm, zeros_hbm, out_hbm,
                sidx, didx, rows, stage, acc, ysh, semg, sems):
    c = lax.axis_index("c")
    s = lax.axis_index("s")
    roff = s * ROWS_PER_TILE
    start = lax.select(c == 0, s * CNT0, NS * CNT0 + s * CNT1)
    ngroup = lax.select(c == 0, CNT0 // GK, CNT1 // GK)

    pltpu.async_copy(src_hbm.at[pl.ds(start, CNTMAX)], sidx, sems)
    pltpu.async_copy(dst_hbm.at[pl.ds(start, CNTMAX)], didx, sems)

    if ysh is not None:
      @pl.when(s < NS - 1)
      def _():
        pltpu.sync_copy(y_hbm.at[pl.ds(s * YCOPY, YCOPY)], stage)
        pltpu.sync_copy(stage, ysh.at[pl.ds(s * YCOPY, YCOPY)])

      @pl.when(s == NS - 1)
      def _():
        rest = N - (NS - 1) * YCOPY
        pltpu.sync_copy(y_hbm.at[pl.ds((NS - 1) * YCOPY, rest)],
                        stage.at[pl.ds(0, rest)])
        pltpu.sync_copy(stage.at[pl.ds(0, rest)],
                        ysh.at[pl.ds((NS - 1) * YCOPY, rest)])

    pltpu.sync_copy(zeros_hbm, stage)
    pltpu.sync_copy(stage, acc.at[pl.ds(roff, ROWS_PER_TILE)])
    pltpu.make_async_copy(src_hbm.at[pl.ds(0, CNTMAX)], sidx, sems).wait()
    pltpu.make_async_copy(src_hbm.at[pl.ds(0, CNTMAX)], didx, sems).wait()
    plsc.subcore_barrier()

    ytab = y_hbm if ysh is None else ysh

    def fire_gathers(g, bank):
      for j in range(GK):
        pltpu.async_copy(ytab.at[sidx.at[g * GK + j]], rows.at[bank, j],
                         semg)

    def drain_gathers(bank):
      for j in range(GK):
        pltpu.make_async_copy(ytab.at[sidx.at[0]], rows.at[bank, j],
                              semg).wait()

    def drain_scatters(bank):
      for j in range(GK):
        pltpu.make_async_copy(rows.at[bank, j], acc.at[didx.at[0]],
                              sems).wait()

    fire_gathers(0, 0)

    def group(g, carry):
      bank = lax.rem(g, 2)
      drain_gathers(bank)

      @pl.when(g + 1 < ngroup)
      def _():
        fire_gathers(g + 1, 1 - bank)

      for j in range(GK):
        pltpu.async_copy(rows.at[bank, j], acc.at[didx.at[g * GK + j]], sems,
                         add=True)
      drain_scatters(bank)
      return carry

    lax.fori_loop(0, ngroup, group, 0)
    plsc.subcore_barrier()

    pltpu.sync_copy(acc.at[pl.ds(roff, ROWS_PER_TILE)], stage)
    pltpu.sync_copy(stage, out_hbm.at[c, pl.ds(roff, ROWS_PER_TILE)])

  return edge_pass


def _relu6(x):
  return jnp.clip(x, 0.0, 6.0)


def _entry_body(feat_ref, w_ref, b_ref, out_ref):
  out_ref[...] = _relu6(
      jnp.dot(feat_ref[...], w_ref[...], preferred_element_type=jnp.float32)
      + b_ref[...])


def _deg_body(degp_ref, x0_ref, w_ref, dinv_ref, y_ref):
  deg = degp_ref[0, :N, 0:1] + degp_ref[1, :N, 0:1] + 1.0
  dinv = lax.rsqrt(jnp.maximum(deg, 1e-12))
  dinv_ref[...] = dinv
  y_ref[...] = jnp.dot(x0_ref[...], w_ref[...],
                       preferred_element_type=jnp.float32) * dinv


def _layer_body(zp_ref, y_ref, dinv_ref, b_ref, wn_ref, yn_ref):
  h = _relu6((zp_ref[0, :N] + zp_ref[1, :N] + y_ref[...]) * dinv_ref[...]
             + b_ref[...])
  yn_ref[...] = jnp.dot(h, wn_ref[...],
                        preferred_element_type=jnp.float32) * dinv_ref[...]


def _head_body(zp_ref, y_ref, dinv_ref, bg3_ref, ws1_ref, bs1_ref, ws2_ref,
               bs2_ref, wt1a_ref, wt1b_ref, bt1_ref, wt2_ref, bt2_ref,
               mask_ref, sprob_ref, sidx_ref, tprob_ref, tidx_ref):
  x = _relu6((zp_ref[0, :N] + zp_ref[1, :N] + y_ref[...]) * dinv_ref[...]
             + bg3_ref[...])
  sh = _relu6(jnp.dot(x, ws1_ref[...], preferred_element_type=jnp.float32)
              + bs1_ref[...])
  sl = jnp.dot(sh, ws2_ref[...], preferred_element_type=jnp.float32) \
      + bs2_ref[...]
  sp = jnp.exp(sl - jnp.max(sl))
  sp = sp / jnp.sum(sp)
  m = mask_ref[...] > 0.0
  sprob_ref[...] = jnp.where(m, 0.0, sp)
  rows = lax.broadcasted_iota(jnp.int32, (N, 1), 0)
  sm = jnp.where(m, -1.0, sp)
  smx = jnp.max(sm)
  sidx = jnp.min(jnp.where(sm == smx, rows, N))
  sidx_ref[...] = jnp.reshape(sidx, (1, 1))
  xs = jnp.sum(jnp.where(rows == sidx, x, 0.0), axis=0, keepdims=True)
  th = _relu6(jnp.dot(x, wt1a_ref[...], preferred_element_type=jnp.float32)
              + jnp.dot(xs, wt1b_ref[...], preferred_element_type=jnp.float32)
              + bt1_ref[...])
  tl = jnp.dot(th, wt2_ref[...], preferred_element_type=jnp.float32) \
      + bt2_ref[...]
  tp = jnp.exp(tl - jnp.max(tl))
  tp = tp / jnp.sum(tp)
  tmask = rows < MAXN
  tprob_ref[...] = jnp.where(tmask, tp, 0.0)
  tmx = jnp.max(jnp.where(tmask, tp, -1.0))
  tidx = jnp.min(jnp.where((tp == tmx) & tmask, rows, N))
  tidx_ref[...] = jnp.reshape(tidx, (1, 1))


def _tc_call(body, out_shapes):
  return pl.pallas_call(
      body,
      out_shape=out_shapes,
  )


def kernel(feat, edge_index, mask_candidate_set, W0, b0, Wg1, bg1, Wg2, bg2,
           Wg3, bg3, Ws1, bs1, Ws2, bs2, Wt1, bt1, Wt2, bt2):
  f32 = jnp.float32
  src = jnp.concatenate(
      [edge_index[0].astype(jnp.int32),
       jnp.zeros((EPAD - E,), jnp.int32)]).reshape(EPAD // CHUNK, CHUNK)
  dst = jnp.concatenate(
      [edge_index[1].astype(jnp.int32),
       jnp.full((EPAD - E,), N, jnp.int32)]).reshape(EPAD // CHUNK, CHUNK)

  x0 = _tc_call(_entry_body, jax.ShapeDtypeStruct((N, 8), f32))(
      feat, W0, b0.reshape(1, 8))

  ones8 = jnp.ones((N, 8), f32)
  degp = _make_edge_pass(8)(ones8, src, dst, jnp.zeros((ROWS_PER_TILE, 8), f32))

  dinv, y1 = _tc_call(
      _deg_body,
      (jax.ShapeDtypeStruct((N, 1), f32), jax.ShapeDtypeStruct((N, 16), f32)),
  )(degp, x0, Wg1)

  z1 = _make_edge_pass(16)(y1, src, dst, jnp.zeros((ROWS_PER_TILE, 16), f32))
  y2 = _tc_call(_layer_body, jax.ShapeDtypeStruct((N, 24), f32))(
      z1, y1, dinv, bg1.reshape(1, 16), Wg2)

  z2 = _make_edge_pass(24)(y2, src, dst, jnp.zeros((ROWS_PER_TILE, 24), f32))
  y3 = _tc_call(_layer_body, jax.ShapeDtypeStruct((N, 32), f32))(
      z2, y2, dinv, bg2.reshape(1, 24), Wg3)

  z3 = _make_edge_pass(32)(y3, src, dst, jnp.zeros((ROWS_PER_TILE, 32), f32))

  sprob, sidx, tprob, tidx = _tc_call(
      _head_body,
      (jax.ShapeDtypeStruct((N, 1), f32),
       jax.ShapeDtypeStruct((1, 1), jnp.int32),
       jax.ShapeDtypeStruct((N, 1), f32),
       jax.ShapeDtypeStruct((1, 1), jnp.int32)),
  )(z3, y3, dinv, bg3.reshape(1, 32), Ws1, bs1.reshape(1, 16), Ws2,
    bs2.reshape(1, 1), Wt1[:32], Wt1[32:], bt1.reshape(1, 24), Wt2,
    bt2.reshape(1, 1), mask_candidate_set.astype(f32).reshape(N, 1))

  return sprob, sidx[0, 0], tprob, tidx[0, 0]


# speedup vs baseline: 1.3765x; 1.0618x over previous
import functools

import jax
import jax.numpy as jnp
from jax import lax
from jax.experimental import pallas as pl
from jax.experimental.pallas import tpu as pltpu
from jax.experimental.pallas import tpu_sc as plsc

N = 10000
E = 160000
MAXN = 9993

NC = 2
NS = 16
NW = NC * NS
CHUNK = 128
NCHUNK = E // CHUNK
NPAD = 10240
ROWS_PER_TILE = NPAD // NS
GK = 8
CHUNKS_PER_TILE = 40
EPAD = NW * CHUNKS_PER_TILE * CHUNK
CNT0, CNT1 = 40, 40
CNTMAX = max(CNT0, CNT1)
YCOPY = 640


def _make_edge_pass(F):
  mesh = plsc.VectorSubcoreMesh(core_axis_name="c", subcore_axis_name="s")

  @functools.partial(
      pl.kernel,
      mesh=mesh,
      out_type=jax.ShapeDtypeStruct((NC, NPAD, F), jnp.float32),
      scratch_types=[
          pltpu.VMEM((CNTMAX, CHUNK), jnp.int32),
          pltpu.VMEM((CNTMAX, CHUNK), jnp.int32),
          pltpu.VMEM((2, GK, CHUNK, F), jnp.float32),
          pltpu.VMEM((ROWS_PER_TILE, F), jnp.float32),
          pltpu.VMEM_SHARED((NPAD, F), jnp.float32),
          pltpu.VMEM_SHARED((N, F), jnp.float32) if F <= 24 else None,
          pltpu.SemaphoreType.DMA,
          pltpu.SemaphoreType.DMA,
      ],
      compiler_params=pltpu.CompilerParams(use_tc_tiling_on_sc=False),
  )
  def edge_pass(y_hbm, src_hbm, dst_hbm, zeros_hbm, out_hbm,
                sidx, didx, rows, stage, acc, ysh, semg, sems):
    c = lax.axis_index("c")
    s = lax.axis_index("s")
    roff = s * ROWS_PER_TILE
    start = lax.select(c == 0, s * CNT0, NS * CNT0 + s * CNT1)
    ngroup = lax.select(c == 0, CNT0 // GK, CNT1 // GK)

    pltpu.async_copy(src_hbm.at[pl.ds(start, CNTMAX)], sidx, sems)
    pltpu.async_copy(dst_hbm.at[pl.ds(start, CNTMAX)], didx, sems)

    if ysh is not None:
      @pl.when(s < NS - 1)
      def _():
        pltpu.sync_copy(y_hbm.at[pl.ds(s * YCOPY, YCOPY)], stage)
        pltpu.sync_copy(stage, ysh.at[pl.ds(s * YCOPY, YCOPY)])

      @pl.when(s == NS - 1)
      def _():
        rest = N - (NS - 1) * YCOPY
        pltpu.sync_copy(y_hbm.at[pl.ds((NS - 1) * YCOPY, rest)],
                        stage.at[pl.ds(0, rest)])
        pltpu.sync_copy(stage.at[pl.ds(0, rest)],
                        ysh.at[pl.ds((NS - 1) * YCOPY, rest)])

    pltpu.sync_copy(zeros_hbm, stage)
    pltpu.sync_copy(stage, acc.at[pl.ds(roff, ROWS_PER_TILE)])
    pltpu.make_async_copy(src_hbm.at[pl.ds(0, CNTMAX)], sidx, sems).wait()
    pltpu.make_async_copy(src_hbm.at[pl.ds(0, CNTMAX)], didx, sems).wait()
    plsc.subcore_barrier()

    ytab = y_hbm if ysh is None else ysh

    def fire_gathers(g, bank):
      for j in range(GK):
        pltpu.async_copy(ytab.at[sidx.at[g * GK + j]], rows.at[bank, j],
                         semg)

    def drain_gathers(bank):
      for j in range(GK):
        pltpu.make_async_copy(ytab.at[sidx.at[0]], rows.at[bank, j],
                              semg).wait()

    def drain_scatters(bank):
      for j in range(GK):
        pltpu.make_async_copy(rows.at[bank, j], acc.at[didx.at[0]],
                              sems).wait()

    fire_gathers(0, 0)

    def group(g, carry):
      bank = lax.rem(g, 2)
      drain_gathers(bank)

      @pl.when(g + 1 < ngroup)
      def _():
        fire_gathers(g + 1, 1 - bank)

      for j in range(GK):
        pltpu.async_copy(rows.at[bank, j], acc.at[didx.at[g * GK + j]], sems,
                         add=True)
      drain_scatters(bank)
      return carry

    lax.fori_loop(0, ngroup, group, 0)
    plsc.subcore_barrier()

    pltpu.sync_copy(acc.at[pl.ds(roff, ROWS_PER_TILE)], stage)
    pltpu.sync_copy(stage, out_hbm.at[c, pl.ds(roff, ROWS_PER_TILE)])

  return edge_pass


def _relu6(x):
  return jnp.clip(x, 0.0, 6.0)


def _entry_body(feat_ref, w_ref, b_ref, out_ref):
  out_ref[...] = _relu6(
      jnp.dot(feat_ref[...], w_ref[...], preferred_element_type=jnp.float32)
      + b_ref[...])


def _deg_body(degp_ref, x0_ref, w_ref, dinv_ref, y_ref):
  deg = degp_ref[0, :N, 0:1] + degp_ref[1, :N, 0:1] + 1.0
  dinv = lax.rsqrt(jnp.maximum(deg, 1e-12))
  dinv_ref[...] = dinv
  y_ref[...] = jnp.dot(x0_ref[...], w_ref[...],
                       preferred_element_type=jnp.float32) * dinv


def _layer_body(zp_ref, y_ref, dinv_ref, b_ref, wn_ref, yn_ref):
  h = _relu6((zp_ref[0, :N] + zp_ref[1, :N] + y_ref[...]) * dinv_ref[...]
             + b_ref[...])
  yn_ref[...] = jnp.dot(h, wn_ref[...],
                        preferred_element_type=jnp.float32) * dinv_ref[...]


def _layer3_body(zp_ref, y_ref, dinv_ref, b_ref, wn_ref, yna_ref, ynb_ref):
  h = _relu6((zp_ref[0, :N] + zp_ref[1, :N] + y_ref[...]) * dinv_ref[...]
             + b_ref[...])
  yn = jnp.dot(h, wn_ref[...],
               preferred_element_type=jnp.float32) * dinv_ref[...]
  yna_ref[...] = yn[:, :16]
  ynb_ref[...] = yn[:, 16:]


def _combine3_body(zpa_ref, zpb_ref, ya_ref, yb_ref, dinv_ref, bg3_ref,
                   x_ref):
  z = jnp.concatenate(
      [zpa_ref[0, :N] + zpa_ref[1, :N] + ya_ref[...],
       zpb_ref[0, :N] + zpb_ref[1, :N] + yb_ref[...]], axis=1)
  x_ref[...] = _relu6(z * dinv_ref[...] + bg3_ref[...])


def _head_body(x_ref, ws1_ref, bs1_ref, ws2_ref, bs2_ref, wt1a_ref, wt1b_ref,
               bt1_ref, wt2_ref, bt2_ref, mask_ref, sprob_ref, sidx_ref,
               tprob_ref, tidx_ref):
  x = x_ref[...]
  sh = _relu6(jnp.dot(x, ws1_ref[...], preferred_element_type=jnp.float32)
              + bs1_ref[...])
  sl = jnp.dot(sh, ws2_ref[...], preferred_element_type=jnp.float32) \
      + bs2_ref[...]
  sp = jnp.exp(sl - jnp.max(sl))
  sp = sp / jnp.sum(sp)
  m = mask_ref[...] > 0.0
  sprob_ref[...] = jnp.where(m, 0.0, sp)
  rows = lax.broadcasted_iota(jnp.int32, (N, 1), 0)
  sm = jnp.where(m, -1.0, sp)
  smx = jnp.max(sm)
  sidx = jnp.min(jnp.where(sm == smx, rows, N))
  sidx_ref[...] = jnp.reshape(sidx, (1, 1))
  xs = jnp.sum(jnp.where(rows == sidx, x, 0.0), axis=0, keepdims=True)
  th = _relu6(jnp.dot(x, wt1a_ref[...], preferred_element_type=jnp.float32)
              + jnp.dot(xs, wt1b_ref[...], preferred_element_type=jnp.float32)
              + bt1_ref[...])
  tl = jnp.dot(th, wt2_ref[...], preferred_element_type=jnp.float32) \
      + bt2_ref[...]
  tp = jnp.exp(tl - jnp.max(tl))
  tp = tp / jnp.sum(tp)
  tmask = rows < MAXN
  tprob_ref[...] = jnp.where(tmask, tp, 0.0)
  tmx = jnp.max(jnp.where(tmask, tp, -1.0))
  tidx = jnp.min(jnp.where((tp == tmx) & tmask, rows, N))
  tidx_ref[...] = jnp.reshape(tidx, (1, 1))


def _tc_call(body, out_shapes):
  return pl.pallas_call(
      body,
      out_shape=out_shapes,
  )


def kernel(feat, edge_index, mask_candidate_set, W0, b0, Wg1, bg1, Wg2, bg2,
           Wg3, bg3, Ws1, bs1, Ws2, bs2, Wt1, bt1, Wt2, bt2):
  f32 = jnp.float32
  src = jnp.concatenate(
      [edge_index[0].astype(jnp.int32),
       jnp.zeros((EPAD - E,), jnp.int32)]).reshape(EPAD // CHUNK, CHUNK)
  dst = jnp.concatenate(
      [edge_index[1].astype(jnp.int32),
       jnp.full((EPAD - E,), N, jnp.int32)]).reshape(EPAD // CHUNK, CHUNK)

  x0 = _tc_call(_entry_body, jax.ShapeDtypeStruct((N, 8), f32))(
      feat, W0, b0.reshape(1, 8))

  ones8 = jnp.ones((N, 8), f32)
  degp = _make_edge_pass(8)(ones8, src, dst, jnp.zeros((ROWS_PER_TILE, 8), f32))

  dinv, y1 = _tc_call(
      _deg_body,
      (jax.ShapeDtypeStruct((N, 1), f32), jax.ShapeDtypeStruct((N, 16), f32)),
  )(degp, x0, Wg1)

  edge16 = _make_edge_pass(16)
  zeros16 = jnp.zeros((ROWS_PER_TILE, 16), f32)
  z1 = edge16(y1, src, dst, zeros16)
  y2 = _tc_call(_layer_body, jax.ShapeDtypeStruct((N, 24), f32))(
      z1, y1, dinv, bg1.reshape(1, 16), Wg2)

  z2 = _make_edge_pass(24)(y2, src, dst, jnp.zeros((ROWS_PER_TILE, 24), f32))
  y3a, y3b = _tc_call(
      _layer3_body,
      (jax.ShapeDtypeStruct((N, 16), f32), jax.ShapeDtypeStruct((N, 16), f32)),
  )(z2, y2, dinv, bg2.reshape(1, 24), Wg3)

  z3a = edge16(y3a, src, dst, zeros16)
  z3b = edge16(y3b, src, dst, zeros16)

  x = _tc_call(_combine3_body, jax.ShapeDtypeStruct((N, 32), f32))(
      z3a, z3b, y3a, y3b, dinv, bg3.reshape(1, 32))

  sprob, sidx, tprob, tidx = _tc_call(
      _head_body,
      (jax.ShapeDtypeStruct((N, 1), f32),
       jax.ShapeDtypeStruct((1, 1), jnp.int32),
       jax.ShapeDtypeStruct((N, 1), f32),
       jax.ShapeDtypeStruct((1, 1), jnp.int32)),
  )(x, Ws1, bs1.reshape(1, 16), Ws2, bs2.reshape(1, 1), Wt1[:32], Wt1[32:],
    bt1.reshape(1, 24), Wt2, bt2.reshape(1, 1),
    mask_candidate_set.astype(f32).reshape(N, 1))

  return sprob, sidx[0, 0], tprob, tidx[0, 0]


# speedup vs baseline: 1.4436x; 1.0488x over previous
import functools

import jax
import jax.numpy as jnp
from jax import lax
from jax.experimental import pallas as pl
from jax.experimental.pallas import tpu as pltpu
from jax.experimental.pallas import tpu_sc as plsc

N = 10000
E = 160000
MAXN = 9993

NC = 2
NS = 16
NW = NC * NS
CHUNK = 128
NCHUNK = E // CHUNK
NPAD = 10240
ROWS_PER_TILE = NPAD // NS
GK = 8
CHUNKS_PER_TILE = 40
EPAD = NW * CHUNKS_PER_TILE * CHUNK
CNT0, CNT1 = 40, 40
CNTMAX = max(CNT0, CNT1)
YCOPY = 640


def _make_edge_pass(F):
  mesh = plsc.VectorSubcoreMesh(core_axis_name="c", subcore_axis_name="s")

  @functools.partial(
      pl.kernel,
      mesh=mesh,
      out_type=jax.ShapeDtypeStruct((NC, NPAD, F), jnp.float32),
      scratch_types=[
          pltpu.VMEM((CNTMAX, CHUNK), jnp.int32),
          pltpu.VMEM((CNTMAX, CHUNK), jnp.int32),
          pltpu.VMEM((2, GK, CHUNK, F), jnp.float32),
          pltpu.VMEM((ROWS_PER_TILE, F), jnp.float32),
          pltpu.VMEM_SHARED((NPAD, F), jnp.float32),
          pltpu.VMEM_SHARED((N, F), jnp.float32) if F <= 24 else None,
          pltpu.SemaphoreType.DMA,
          pltpu.SemaphoreType.DMA,
      ],
      compiler_params=pltpu.CompilerParams(use_tc_tiling_on_sc=False),
  )
  def edge_pass(y_hbm, src_hbm, dst_hbm, zeros_hbm, out_hbm,
                sidx, didx, rows, stage, acc, ysh, semg, sems):
    c = lax.axis_index("c")
    s = lax.axis_index("s")
    roff = s * ROWS_PER_TILE
    start = lax.select(c == 0, s * CNT0, NS * CNT0 + s * CNT1)
    ngroup = lax.select(c == 0, CNT0 // GK, CNT1 // GK)

    pltpu.async_copy(src_hbm.at[pl.ds(start, CNTMAX)], sidx, sems)
    pltpu.async_copy(dst_hbm.at[pl.ds(start, CNTMAX)], didx, sems)

    if ysh is not None:
      @pl.when(s < NS - 1)
      def _():
        pltpu.sync_copy(y_hbm.at[pl.ds(s * YCOPY, YCOPY)], stage)
        pltpu.sync_copy(stage, ysh.at[pl.ds(s * YCOPY, YCOPY)])

      @pl.when(s == NS - 1)
      def _():
        rest = N - (NS - 1) * YCOPY
        pltpu.sync_copy(y_hbm.at[pl.ds((NS - 1) * YCOPY, rest)],
                        stage.at[pl.ds(0, rest)])
        pltpu.sync_copy(stage.at[pl.ds(0, rest)],
                        ysh.at[pl.ds((NS - 1) * YCOPY, rest)])

    pltpu.sync_copy(zeros_hbm, stage)
    pltpu.sync_copy(stage, acc.at[pl.ds(roff, ROWS_PER_TILE)])
    pltpu.make_async_copy(src_hbm.at[pl.ds(0, CNTMAX)], sidx, sems).wait()
    pltpu.make_async_copy(src_hbm.at[pl.ds(0, CNTMAX)], didx, sems).wait()
    plsc.subcore_barrier()

    ytab = y_hbm if ysh is None else ysh

    def fire_gathers(g, bank):
      for j in range(GK):
        pltpu.async_copy(ytab.at[sidx.at[g * GK + j]], rows.at[bank, j],
                         semg)

    def drain_gathers(bank):
      for j in range(GK):
        pltpu.make_async_copy(ytab.at[sidx.at[0]], rows.at[bank, j],
                              semg).wait()

    def drain_scatters(bank):
      for j in range(GK):
        pltpu.make_async_copy(rows.at[bank, j], acc.at[didx.at[0]],
                              sems).wait()

    fire_gathers(0, 0)

    def group(g, carry):
      bank = lax.rem(g, 2)
      drain_gathers(bank)

      @pl.when(g + 1 < ngroup)
      def _():
        fire_gathers(g + 1, 1 - bank)

      for j in range(GK):
        pltpu.async_copy(rows.at[bank, j], acc.at[didx.at[g * GK + j]], sems,
                         add=True)
      drain_scatters(bank)
      return carry

    lax.fori_loop(0, ngroup, group, 0)
    plsc.subcore_barrier()

    pltpu.sync_copy(acc.at[pl.ds(roff, ROWS_PER_TILE)], stage)
    pltpu.sync_copy(stage, out_hbm.at[c, pl.ds(roff, ROWS_PER_TILE)])

  return edge_pass


F3 = 16
NCHUNK3 = 1280
TILE3 = NCHUNK3 // NS
NG3 = TILE3 // GK


def _make_edge_pass3():
  mesh = plsc.VectorSubcoreMesh(core_axis_name="c", subcore_axis_name="s")

  @functools.partial(
      pl.kernel,
      mesh=mesh,
      out_type=jax.ShapeDtypeStruct((NC, NPAD, F3), jnp.float32),
      scratch_types=[
          pltpu.VMEM((TILE3, CHUNK), jnp.int32),
          pltpu.VMEM((TILE3, CHUNK), jnp.int32),
          pltpu.VMEM((2, GK, CHUNK, F3), jnp.float32),
          pltpu.VMEM((ROWS_PER_TILE, F3), jnp.float32),
          pltpu.VMEM_SHARED((NPAD, F3), jnp.float32),
          pltpu.VMEM_SHARED((N, F3), jnp.float32),
          pltpu.SemaphoreType.DMA,
          pltpu.SemaphoreType.DMA,
      ],
      compiler_params=pltpu.CompilerParams(use_tc_tiling_on_sc=False),
  )
  def edge_pass3(ya_hbm, yb_hbm, src_hbm, dst_hbm, zeros_hbm, out_hbm,
                 sidx, didx, rows, stage, acc, ysh, semg, sems):
    c = lax.axis_index("c")
    s = lax.axis_index("s")
    roff = s * ROWS_PER_TILE
    start = s * TILE3

    pltpu.async_copy(src_hbm.at[pl.ds(start, TILE3)], sidx, sems)
    pltpu.async_copy(dst_hbm.at[pl.ds(start, TILE3)], didx, sems)

    @pl.when(s < NS - 1)
    def _():
      @pl.when(c == 0)
      def _():
        pltpu.sync_copy(ya_hbm.at[pl.ds(s * YCOPY, YCOPY)], stage)

      @pl.when(c == 1)
      def _():
        pltpu.sync_copy(yb_hbm.at[pl.ds(s * YCOPY, YCOPY)], stage)

      pltpu.sync_copy(stage, ysh.at[pl.ds(s * YCOPY, YCOPY)])

    @pl.when(s == NS - 1)
    def _():
      rest = N - (NS - 1) * YCOPY

      @pl.when(c == 0)
      def _():
        pltpu.sync_copy(ya_hbm.at[pl.ds((NS - 1) * YCOPY, rest)],
                        stage.at[pl.ds(0, rest)])

      @pl.when(c == 1)
      def _():
        pltpu.sync_copy(yb_hbm.at[pl.ds((NS - 1) * YCOPY, rest)],
                        stage.at[pl.ds(0, rest)])

      pltpu.sync_copy(stage.at[pl.ds(0, rest)],
                      ysh.at[pl.ds((NS - 1) * YCOPY, rest)])

    pltpu.sync_copy(zeros_hbm, stage)
    pltpu.sync_copy(stage, acc.at[pl.ds(roff, ROWS_PER_TILE)])
    pltpu.make_async_copy(src_hbm.at[pl.ds(0, TILE3)], sidx, sems).wait()
    pltpu.make_async_copy(src_hbm.at[pl.ds(0, TILE3)], didx, sems).wait()
    plsc.subcore_barrier()

    def fire_gathers(g, bank):
      for j in range(GK):
        pltpu.async_copy(ysh.at[sidx.at[g * GK + j]], rows.at[bank, j], semg)

    def drain_gathers(bank):
      for j in range(GK):
        pltpu.make_async_copy(ysh.at[sidx.at[0]], rows.at[bank, j],
                              semg).wait()

    def drain_scatters(bank):
      for j in range(GK):
        pltpu.make_async_copy(rows.at[bank, j], acc.at[didx.at[0]],
                              sems).wait()

    fire_gathers(0, 0)

    def group(g, carry):
      bank = lax.rem(g, 2)
      drain_gathers(bank)

      @pl.when(g + 1 < NG3)
      def _():
        fire_gathers(g + 1, 1 - bank)

      for j in range(GK):
        pltpu.async_copy(rows.at[bank, j], acc.at[didx.at[g * GK + j]], sems,
                         add=True)
      drain_scatters(bank)
      return carry

    lax.fori_loop(0, NG3, group, 0)
    plsc.subcore_barrier()

    pltpu.sync_copy(acc.at[pl.ds(roff, ROWS_PER_TILE)], stage)
    pltpu.sync_copy(stage, out_hbm.at[c, pl.ds(roff, ROWS_PER_TILE)])

  return edge_pass3


def _relu6(x):
  return jnp.clip(x, 0.0, 6.0)


def _entry_body(feat_ref, w_ref, b_ref, out_ref):
  out_ref[...] = _relu6(
      jnp.dot(feat_ref[...], w_ref[...], preferred_element_type=jnp.float32)
      + b_ref[...])


def _deg_body(degp_ref, x0_ref, w_ref, dinv_ref, y_ref):
  deg = degp_ref[0, :N, 0:1] + degp_ref[1, :N, 0:1] + 1.0
  dinv = lax.rsqrt(jnp.maximum(deg, 1e-12))
  dinv_ref[...] = dinv
  y_ref[...] = jnp.dot(x0_ref[...], w_ref[...],
                       preferred_element_type=jnp.float32) * dinv


def _layer_body(zp_ref, y_ref, dinv_ref, b_ref, wn_ref, yn_ref):
  h = _relu6((zp_ref[0, :N] + zp_ref[1, :N] + y_ref[...]) * dinv_ref[...]
             + b_ref[...])
  yn_ref[...] = jnp.dot(h, wn_ref[...],
                        preferred_element_type=jnp.float32) * dinv_ref[...]


def _layer3_body(zp_ref, y_ref, dinv_ref, b_ref, wn_ref, yna_ref, ynb_ref):
  h = _relu6((zp_ref[0, :N] + zp_ref[1, :N] + y_ref[...]) * dinv_ref[...]
             + b_ref[...])
  yn = jnp.dot(h, wn_ref[...],
               preferred_element_type=jnp.float32) * dinv_ref[...]
  yna_ref[...] = yn[:, :16]
  ynb_ref[...] = yn[:, 16:]


def _combine3_body(z3_ref, ya_ref, yb_ref, dinv_ref, bg3_ref, x_ref):
  z = jnp.concatenate(
      [z3_ref[0, :N] + ya_ref[...], z3_ref[1, :N] + yb_ref[...]], axis=1)
  x_ref[...] = _relu6(z * dinv_ref[...] + bg3_ref[...])


def _head_body(x_ref, ws1_ref, bs1_ref, ws2_ref, bs2_ref, wt1a_ref, wt1b_ref,
               bt1_ref, wt2_ref, bt2_ref, mask_ref, sprob_ref, sidx_ref,
               tprob_ref, tidx_ref):
  x = x_ref[...]
  sh = _relu6(jnp.dot(x, ws1_ref[...], preferred_element_type=jnp.float32)
              + bs1_ref[...])
  sl = jnp.dot(sh, ws2_ref[...], preferred_element_type=jnp.float32) \
      + bs2_ref[...]
  sp = jnp.exp(sl - jnp.max(sl))
  sp = sp / jnp.sum(sp)
  m = mask_ref[...] > 0.0
  sprob_ref[...] = jnp.where(m, 0.0, sp)
  rows = lax.broadcasted_iota(jnp.int32, (N, 1), 0)
  sm = jnp.where(m, -1.0, sp)
  smx = jnp.max(sm)
  sidx = jnp.min(jnp.where(sm == smx, rows, N))
  sidx_ref[...] = jnp.reshape(sidx, (1, 1))
  xs = jnp.sum(jnp.where(rows == sidx, x, 0.0), axis=0, keepdims=True)
  th = _relu6(jnp.dot(x, wt1a_ref[...], preferred_element_type=jnp.float32)
              + jnp.dot(xs, wt1b_ref[...], preferred_element_type=jnp.float32)
              + bt1_ref[...])
  tl = jnp.dot(th, wt2_ref[...], preferred_element_type=jnp.float32) \
      + bt2_ref[...]
  tp = jnp.exp(tl - jnp.max(tl))
  tp = tp / jnp.sum(tp)
  tmask = rows < MAXN
  tprob_ref[...] = jnp.where(tmask, tp, 0.0)
  tmx = jnp.max(jnp.where(tmask, tp, -1.0))
  tidx = jnp.min(jnp.where((tp == tmx) & tmask, rows, N))
  tidx_ref[...] = jnp.reshape(tidx, (1, 1))


def _tc_call(body, out_shapes):
  return pl.pallas_call(
      body,
      out_shape=out_shapes,
  )


def kernel(feat, edge_index, mask_candidate_set, W0, b0, Wg1, bg1, Wg2, bg2,
           Wg3, bg3, Ws1, bs1, Ws2, bs2, Wt1, bt1, Wt2, bt2):
  f32 = jnp.float32
  src = jnp.concatenate(
      [edge_index[0].astype(jnp.int32),
       jnp.zeros((EPAD - E,), jnp.int32)]).reshape(EPAD // CHUNK, CHUNK)
  dst = jnp.concatenate(
      [edge_index[1].astype(jnp.int32),
       jnp.full((EPAD - E,), N, jnp.int32)]).reshape(EPAD // CHUNK, CHUNK)

  x0 = _tc_call(_entry_body, jax.ShapeDtypeStruct((N, 8), f32))(
      feat, W0, b0.reshape(1, 8))

  ones8 = jnp.ones((N, 8), f32)
  degp = _make_edge_pass(8)(ones8, src, dst, jnp.zeros((ROWS_PER_TILE, 8), f32))

  dinv, y1 = _tc_call(
      _deg_body,
      (jax.ShapeDtypeStruct((N, 1), f32), jax.ShapeDtypeStruct((N, 16), f32)),
  )(degp, x0, Wg1)

  edge16 = _make_edge_pass(16)
  zeros16 = jnp.zeros((ROWS_PER_TILE, 16), f32)
  z1 = edge16(y1, src, dst, zeros16)
  y2 = _tc_call(_layer_body, jax.ShapeDtypeStruct((N, 24), f32))(
      z1, y1, dinv, bg1.reshape(1, 16), Wg2)

  z2 = _make_edge_pass(24)(y2, src, dst, jnp.zeros((ROWS_PER_TILE, 24), f32))
  y3a, y3b = _tc_call(
      _layer3_body,
      (jax.ShapeDtypeStruct((N, 16), f32), jax.ShapeDtypeStruct((N, 16), f32)),
  )(z2, y2, dinv, bg2.reshape(1, 24), Wg3)

  z3 = _make_edge_pass3()(y3a, y3b, src, dst, zeros16)

  x = _tc_call(_combine3_body, jax.ShapeDtypeStruct((N, 32), f32))(
      z3, y3a, y3b, dinv, bg3.reshape(1, 32))

  sprob, sidx, tprob, tidx = _tc_call(
      _head_body,
      (jax.ShapeDtypeStruct((N, 1), f32),
       jax.ShapeDtypeStruct((1, 1), jnp.int32),
       jax.ShapeDtypeStruct((N, 1), f32),
       jax.ShapeDtypeStruct((1, 1), jnp.int32)),
  )(x, Ws1, bs1.reshape(1, 16), Ws2, bs2.reshape(1, 1), Wt1[:32], Wt1[32:],
    bt1.reshape(1, 24), Wt2, bt2.reshape(1, 1),
    mask_candidate_set.astype(f32).reshape(N, 1))

  return sprob, sidx[0, 0], tprob, tidx[0, 0]
